# Initial kernel scaffold; baseline (speedup 1.0000x reference)
import jax
import jax.numpy as jnp
from jax.experimental import pallas as pl


def kernel(poi_embs, edge_index, edge_attr, W0, b0, W1, b1):
    raise NotImplementedError("write your pallas kernel here")


# speedup vs baseline: 5.6286x; 5.6286x over previous
import functools

import jax
import jax.numpy as jnp
from jax import lax
from jax.experimental import pallas as pl
from jax.experimental.pallas import tpu as pltpu
from jax.experimental.pallas import tpu_sc as plsc

N = 10000
D = 256
H = 128
E = 160000
NC = 2
NS = 16
NPAD = 10240
K = 80
EPAD = 163840
EPT = EPAD // NS
NCH = EPT // K
APT = E // NS
ROWS_PT = NPAD // NS
ZROWS = 128
SLOPE = 0.01
BM = 1000

_sc_mesh = plsc.VectorSubcoreMesh(core_axis_name="c", subcore_axis_name="s")


@functools.partial(
    pl.kernel,
    out_type=(
        jax.ShapeDtypeStruct((NPAD,), jnp.float32),
        jax.ShapeDtypeStruct((E,), jnp.float32),
    ),
    mesh=_sc_mesh,
    scratch_types=[
        pltpu.VMEM((NCH, K), jnp.int32),
        pltpu.VMEM((K,), jnp.float32),
        pltpu.VMEM((APT,), jnp.float32),
        pltpu.VMEM((NPAD // NS,), jnp.float32),
        pltpu.VMEM_SHARED((NPAD,), jnp.float32),
    ],
)
def _pre_kernel(col_hbm, attr_hbm, deg_hbm, dist_hbm,
                col_v, ones_v, fbuf, zbuf, deg_sh):
    c = lax.axis_index("c")
    s = lax.axis_index("s")

    @pl.when(c == 1)
    def _dist():
        base = s * APT
        pltpu.sync_copy(attr_hbm.at[pl.ds(base, APT)], fbuf)

        def body(i, carry):
            v = fbuf[pl.ds(i * 16, 16)]
            fbuf[pl.ds(i * 16, 16)] = jnp.exp(-(v * v))
            return carry

        lax.fori_loop(0, APT // 16, body, 0)
        pltpu.sync_copy(fbuf, dist_hbm.at[pl.ds(base, APT)])

    @pl.when(c == 0)
    def _deg():
        zc = NPAD // NS

        def zb(i, carry):
            zbuf[pl.ds(i * 16, 16)] = jnp.zeros((16,), jnp.float32)
            return carry

        lax.fori_loop(0, zc // 16, zb, 0)

        def ob(i, carry):
            ones_v[pl.ds(i * 16, 16)] = jnp.ones((16,), jnp.float32)
            return carry

        lax.fori_loop(0, K // 16, ob, 0)
        pltpu.sync_copy(zbuf, deg_sh.at[pl.ds(s * zc, zc)])
        pltpu.sync_copy(col_hbm.at[pl.ds(s * NCH, NCH)], col_v)
        plsc.subcore_barrier()

        def sb(j, carry):
            pltpu.sync_copy(ones_v, deg_sh.at[col_v.at[j]], add=True)
            return carry

        lax.fori_loop(0, NCH, sb, 0)
        plsc.subcore_barrier()

        @pl.when(s == 0)
        def _out():
            pltpu.sync_copy(deg_sh, deg_hbm)


@functools.partial(
    pl.kernel,
    out_type=jax.ShapeDtypeStruct((NC * NPAD, H), jnp.float32),
    mesh=_sc_mesh,
    scratch_types=[
        pltpu.VMEM((EPT,), jnp.int32),
        pltpu.VMEM((NCH, K), jnp.int32),
        pltpu.VMEM((EPT,), jnp.float32),
        pltpu.VMEM((K, H), jnp.float32),
        pltpu.VMEM_SHARED((NPAD, H), jnp.float32),
        pltpu.SemaphoreType.DMA,
    ],
)
def _scatter_kernel(xs_hbm, row_hbm, col_hbm, dist_hbm, out_hbm,
                    row_v, col_v, dist_v, buf, acc_sh, sem):
    c = lax.axis_index("c")
    s = lax.axis_index("s")
    ebase = s * EPT
    pltpu.sync_copy(row_hbm.at[pl.ds(ebase, EPT)], row_v)
    pltpu.sync_copy(col_hbm.at[pl.ds(s * NCH, NCH)], col_v)
    pltpu.sync_copy(dist_hbm.at[pl.ds(ebase, EPT)], dist_v)

    off = c * N

    def adj(i, carry):
        row_v[pl.ds(i * 16, 16)] = row_v[pl.ds(i * 16, 16)] + off
        return carry

    lax.fori_loop(0, EPT // 16, adj, 0)

    def zb(i, carry):
        for v in range(H // 16):
            buf[i, pl.ds(v * 16, 16)] = jnp.zeros((16,), jnp.float32)
        return carry

    lax.fori_loop(0, K, zb, 0)
    for t in range(ROWS_PT // K):
        pltpu.sync_copy(buf, acc_sh.at[pl.ds(s * ROWS_PT + t * K, K)])
    plsc.subcore_barrier()

    def chunk(j, carry):
        pltpu.async_copy(xs_hbm.at[row_v.at[pl.ds(j * K, K)]], buf, sem).wait()

        def grp(g, carry2):
            dvec = dist_v[pl.ds(j * K + g * 16, 16)]
            for e in range(16):
                d = dvec[e]
                i = g * 16 + e
                for v in range(H // 16):
                    buf[i, pl.ds(v * 16, 16)] = buf[i, pl.ds(v * 16, 16)] * d
            return carry2

        lax.fori_loop(0, K // 16, grp, 0)
        pltpu.sync_copy(buf, acc_sh.at[col_v.at[j]], add=True)
        return carry

    lax.fori_loop(0, NCH, chunk, 0)
    plsc.subcore_barrier()
    obase = c * NPAD + s * ROWS_PT
    pltpu.sync_copy(acc_sh.at[pl.ds(s * ROWS_PT, ROWS_PT)],
                    out_hbm.at[pl.ds(obase, ROWS_PT)])


def _mm0_body(deg_ref, x_ref, w_ref, b_ref, dinv_ref, xlin_ref, xsp_ref):
    dinv = lax.rsqrt(deg_ref[...] + 1.0)
    xlin = lax.dot_general(x_ref[...], w_ref[...], (((1,), (1,)), ((), ())),
                           preferred_element_type=jnp.float32) + b_ref[...]
    dinv_ref[...] = dinv
    xlin_ref[...] = xlin
    xs = xlin * dinv
    xsp_ref[0] = xs[:, :H]
    xsp_ref[1] = xs[:, H:]


_mm0 = pl.pallas_call(
    _mm0_body,
    grid=(N // BM,),
    in_specs=[
        pl.BlockSpec((BM, 1), lambda i: (i, 0)),
        pl.BlockSpec((BM, D), lambda i: (i, 0)),
        pl.BlockSpec((D, D), lambda i: (0, 0)),
        pl.BlockSpec((1, D), lambda i: (0, 0)),
    ],
    out_specs=[
        pl.BlockSpec((BM, 1), lambda i: (i, 0)),
        pl.BlockSpec((BM, D), lambda i: (i, 0)),
        pl.BlockSpec((NC, BM, H), lambda i: (0, i, 0)),
    ],
    out_shape=[
        jax.ShapeDtypeStruct((N, 1), jnp.float32),
        jax.ShapeDtypeStruct((N, D), jnp.float32),
        jax.ShapeDtypeStruct((NC, N, H), jnp.float32),
    ],
)


def _mid_body(xlin_ref, accp_ref, dinv_ref, w_ref, b_ref,
              h_ref, xlin1_ref, xsp_ref):
    dinv = dinv_ref[...]
    acc = jnp.concatenate([accp_ref[0], accp_ref[1]], axis=1)
    conv = dinv * acc + (dinv * dinv) * xlin_ref[...]
    h = jnp.where(conv >= 0, conv, SLOPE * conv)
    h_ref[...] = h
    xlin1 = lax.dot_general(h, w_ref[...], (((1,), (1,)), ((), ())),
                            preferred_element_type=jnp.float32) + b_ref[...]
    xlin1_ref[...] = xlin1
    xs = xlin1 * dinv
    xsp_ref[0] = xs[:, :H]
    xsp_ref[1] = xs[:, H:]


_mid = pl.pallas_call(
    _mid_body,
    grid=(N // BM,),
    in_specs=[
        pl.BlockSpec((BM, D), lambda i: (i, 0)),
        pl.BlockSpec((NC, BM, H), lambda i: (0, i, 0)),
        pl.BlockSpec((BM, 1), lambda i: (i, 0)),
        pl.BlockSpec((D, D), lambda i: (0, 0)),
        pl.BlockSpec((1, D), lambda i: (0, 0)),
    ],
    out_specs=[
        pl.BlockSpec((BM, D), lambda i: (i, 0)),
        pl.BlockSpec((BM, D), lambda i: (i, 0)),
        pl.BlockSpec((NC, BM, H), lambda i: (0, i, 0)),
    ],
    out_shape=[
        jax.ShapeDtypeStruct((N, D), jnp.float32),
        jax.ShapeDtypeStruct((N, D), jnp.float32),
        jax.ShapeDtypeStruct((NC, N, H), jnp.float32),
    ],
)


def _fin_body(x0_ref, h_ref, xlin_ref, accp_ref, dinv_ref, out_ref):
    dinv = dinv_ref[...]
    acc = jnp.concatenate([accp_ref[0], accp_ref[1]], axis=1)
    conv = dinv * acc + (dinv * dinv) * xlin_ref[...]
    h2 = jnp.where(conv >= 0, conv, SLOPE * conv)
    out_ref[...] = (x0_ref[...] + h_ref[...] + h2) * (1.0 / 3.0)


_fin = pl.pallas_call(
    _fin_body,
    grid=(N // BM,),
    in_specs=[
        pl.BlockSpec((BM, D), lambda i: (i, 0)),
        pl.BlockSpec((BM, D), lambda i: (i, 0)),
        pl.BlockSpec((BM, D), lambda i: (i, 0)),
        pl.BlockSpec((NC, BM, H), lambda i: (0, i, 0)),
        pl.BlockSpec((BM, 1), lambda i: (i, 0)),
    ],
    out_specs=pl.BlockSpec((BM, D), lambda i: (i, 0)),
    out_shape=jax.ShapeDtypeStruct((N, D), jnp.float32),
)


def kernel(poi_embs, edge_index, edge_attr, W0, b0, W1, b1):
    row = edge_index[0]
    col = edge_index[1]
    pe = EPAD - E
    row_p = jnp.concatenate([row, jnp.zeros((pe,), jnp.int32)])
    col_main = jnp.concatenate([col, jnp.zeros((pe,), jnp.int32)]).reshape(EPAD // K, K)
    col_deg = jnp.concatenate([col, jnp.full((pe,), NPAD - 1, jnp.int32)]).reshape(EPAD // K, K)
    b0r = b0.reshape(1, D)
    b1r = b1.reshape(1, D)

    deg, dist = _pre_kernel(col_deg, edge_attr)
    dist_p = jnp.concatenate([dist, jnp.zeros((pe,), jnp.float32)])
    degc = deg[:N].reshape(N, 1)

    dinv, xlin0, xsp0 = _mm0(degc, poi_embs, W0, b0r)
    acc0 = _scatter_kernel(xsp0.reshape(NC * N, H), row_p, col_main, dist_p)
    h1, xlin1, xsp1 = _mid(xlin0, acc0.reshape(NC, NPAD, H), dinv, W1, b1r)
    acc1 = _scatter_kernel(xsp1.reshape(NC * N, H), row_p, col_main, dist_p)
    return _fin(poi_embs, h1, xlin1, acc1.reshape(NC, NPAD, H), dinv)


# speedup vs baseline: 8.0548x; 1.4310x over previous
import functools

import jax
import jax.numpy as jnp
from jax import lax
from jax.experimental import pallas as pl
from jax.experimental.pallas import tpu as pltpu
from jax.experimental.pallas import tpu_sc as plsc

N = 10000
D = 256
H = 128
E = 160000
NC = 2
NS = 16
NPAD = 10240
K = 128
EPAD = 163840
EPT = EPAD // NS
NCH = EPT // K
APT = E // NS
ROWS_PT = NPAD // NS
ZROWS = 128
SLOPE = 0.01
BM = 1000

_sc_mesh = plsc.VectorSubcoreMesh(core_axis_name="c", subcore_axis_name="s")


@functools.partial(
    pl.kernel,
    out_type=(
        jax.ShapeDtypeStruct((NPAD,), jnp.float32),
        jax.ShapeDtypeStruct((E,), jnp.float32),
    ),
    mesh=_sc_mesh,
    scratch_types=[
        pltpu.VMEM((NCH, K), jnp.int32),
        pltpu.VMEM((K,), jnp.float32),
        pltpu.VMEM((APT,), jnp.float32),
        pltpu.VMEM((NPAD // NS,), jnp.float32),
        pltpu.VMEM_SHARED((NPAD,), jnp.float32),
    ],
)
def _pre_kernel(col_hbm, attr_hbm, deg_hbm, dist_hbm,
                col_v, ones_v, fbuf, zbuf, deg_sh):
    c = lax.axis_index("c")
    s = lax.axis_index("s")

    @pl.when(c == 1)
    def _dist():
        base = s * APT
        pltpu.sync_copy(attr_hbm.at[pl.ds(base, APT)], fbuf)

        def body(i, carry):
            v = fbuf[pl.ds(i * 16, 16)]
            fbuf[pl.ds(i * 16, 16)] = jnp.exp(-(v * v))
            return carry

        lax.fori_loop(0, APT // 16, body, 0)
        pltpu.sync_copy(fbuf, dist_hbm.at[pl.ds(base, APT)])

    @pl.when(c == 0)
    def _deg():
        zc = NPAD // NS

        def zb(i, carry):
            zbuf[pl.ds(i * 16, 16)] = jnp.zeros((16,), jnp.float32)
            return carry

        lax.fori_loop(0, zc // 16, zb, 0)

        def ob(i, carry):
            ones_v[pl.ds(i * 16, 16)] = jnp.ones((16,), jnp.float32)
            return carry

        lax.fori_loop(0, K // 16, ob, 0)
        pltpu.sync_copy(zbuf, deg_sh.at[pl.ds(s * zc, zc)])
        pltpu.sync_copy(col_hbm.at[pl.ds(s * NCH, NCH)], col_v)
        plsc.subcore_barrier()

        def sb(j, carry):
            pltpu.sync_copy(ones_v, deg_sh.at[col_v.at[j]], add=True)
            return carry

        lax.fori_loop(0, NCH, sb, 0)
        plsc.subcore_barrier()

        @pl.when(s == 0)
        def _out():
            pltpu.sync_copy(deg_sh, deg_hbm)


@functools.partial(
    pl.kernel,
    out_type=jax.ShapeDtypeStruct((NC * NPAD, H), jnp.float32),
    mesh=_sc_mesh,
    scratch_types=(
        [pltpu.VMEM((K, H), jnp.float32) for _ in range(2)]
        + [pltpu.VMEM((3, K), jnp.int32) for _ in range(4)]
        + [pltpu.VMEM_SHARED((NPAD, H), jnp.float32)]
        + [pltpu.SemaphoreType.DMA for _ in range(6)]
    ),
)
def _scatter_kernel(xs_hbm, meta_hbm, out_hbm,
                    g0, g1, m0, m1, m2, m3, acc_sh,
                    gsem0, gsem1, msem0, msem1, msem2, msem3):
    c = lax.axis_index("c")
    s = lax.axis_index("s")
    gbuf = [g0, g1]
    mrec = [m0, m1, m2, m3]
    gsem = [gsem0, gsem1]
    msem = [msem0, msem1, msem2, msem3]
    mbase = s * NCH
    off = c * N

    def mstart(jj, ms):
        pltpu.async_copy(meta_hbm.at[mbase + jj], mrec[ms], msem[ms])

    def mwait(ms):
        pltpu.make_async_copy(meta_hbm.at[mbase], mrec[ms], msem[ms]).wait()

    def adjust(ms):
        for g in range(K // 16):
            mrec[ms][0, pl.ds(g * 16, 16)] = mrec[ms][0, pl.ds(g * 16, 16)] + off

    def gstart(gs, ms):
        pltpu.async_copy(xs_hbm.at[mrec[ms].at[0]], gbuf[gs], gsem[gs])

    def gwait(gs, ms):
        pltpu.make_async_copy(xs_hbm.at[mrec[ms].at[0]], gbuf[gs], gsem[gs]).wait()

    def scale(gs, ms):
        def grp(g, carry):
            dv = lax.bitcast_convert_type(mrec[ms][2, pl.ds(g * 16, 16)], jnp.float32)
            for e in range(16):
                d = dv[e]
                i = g * 16 + e
                for v in range(H // 16):
                    gbuf[gs][i, pl.ds(v * 16, 16)] = gbuf[gs][i, pl.ds(v * 16, 16)] * d
            return carry

        lax.fori_loop(0, K // 16, grp, 0)

    def zb(i, carry):
        for v in range(H // 16):
            g0[i, pl.ds(v * 16, 16)] = jnp.zeros((16,), jnp.float32)
        return carry

    lax.fori_loop(0, K, zb, 0)
    for t in range(ROWS_PT // K):
        pltpu.sync_copy(g0, acc_sh.at[pl.ds(s * ROWS_PT + t * K, K)])

    mstart(0, 0)
    mstart(1, 1)
    mwait(0)
    adjust(0)
    gstart(0, 0)
    plsc.subcore_barrier()

    def body(i, carry):
        for u in range(4):
            j = 4 * i + u
            gs = u % 2

            @pl.when(j + 2 < NCH)
            def _():
                mstart(j + 2, (u + 2) % 4)

            @pl.when(j + 1 < NCH)
            def _():
                mwait((u + 1) % 4)
                adjust((u + 1) % 4)
                gstart((u + 1) % 2, (u + 1) % 4)

            gwait(gs, u)
            scale(gs, u)
            pltpu.sync_copy(gbuf[gs], acc_sh.at[mrec[u].at[1]], add=True)
        return carry

    lax.fori_loop(0, NCH // 4, body, 0)
    plsc.subcore_barrier()
    obase = c * NPAD + s * ROWS_PT
    pltpu.sync_copy(acc_sh.at[pl.ds(s * ROWS_PT, ROWS_PT)],
                    out_hbm.at[pl.ds(obase, ROWS_PT)])


def _mm0_body(deg_ref, x_ref, w_ref, b_ref, dinv_ref, xlin_ref, xsp_ref):
    dinv = lax.rsqrt(deg_ref[...] + 1.0)
    xlin = lax.dot_general(x_ref[...], w_ref[...], (((1,), (1,)), ((), ())),
                           preferred_element_type=jnp.float32) + b_ref[...]
    dinv_ref[...] = dinv
    xlin_ref[...] = xlin
    xs = xlin * dinv
    xsp_ref[0] = xs[:, :H]
    xsp_ref[1] = xs[:, H:]


_mm0 = pl.pallas_call(
    _mm0_body,
    grid=(N // BM,),
    in_specs=[
        pl.BlockSpec((BM, 1), lambda i: (i, 0)),
        pl.BlockSpec((BM, D), lambda i: (i, 0)),
        pl.BlockSpec((D, D), lambda i: (0, 0)),
        pl.BlockSpec((1, D), lambda i: (0, 0)),
    ],
    out_specs=[
        pl.BlockSpec((BM, 1), lambda i: (i, 0)),
        pl.BlockSpec((BM, D), lambda i: (i, 0)),
        pl.BlockSpec((NC, BM, H), lambda i: (0, i, 0)),
    ],
    out_shape=[
        jax.ShapeDtypeStruct((N, 1), jnp.float32),
        jax.ShapeDtypeStruct((N, D), jnp.float32),
        jax.ShapeDtypeStruct((NC, N, H), jnp.float32),
    ],
)


def _mid_body(xlin_ref, accp_ref, dinv_ref, w_ref, b_ref,
              h_ref, xlin1_ref, xsp_ref):
    dinv = dinv_ref[...]
    acc = jnp.concatenate([accp_ref[0], accp_ref[1]], axis=1)
    conv = dinv * acc + (dinv * dinv) * xlin_ref[...]
    h = jnp.where(conv >= 0, conv, SLOPE * conv)
    h_ref[...] = h
    xlin1 = lax.dot_general(h, w_ref[...], (((1,), (1,)), ((), ())),
                            preferred_element_type=jnp.float32) + b_ref[...]
    xlin1_ref[...] = xlin1
    xs = xlin1 * dinv
    xsp_ref[0] = xs[:, :H]
    xsp_ref[1] = xs[:, H:]


_mid = pl.pallas_call(
    _mid_body,
    grid=(N // BM,),
    in_specs=[
        pl.BlockSpec((BM, D), lambda i: (i, 0)),
        pl.BlockSpec((NC, BM, H), lambda i: (0, i, 0)),
        pl.BlockSpec((BM, 1), lambda i: (i, 0)),
        pl.BlockSpec((D, D), lambda i: (0, 0)),
        pl.BlockSpec((1, D), lambda i: (0, 0)),
    ],
    out_specs=[
        pl.BlockSpec((BM, D), lambda i: (i, 0)),
        pl.BlockSpec((BM, D), lambda i: (i, 0)),
        pl.BlockSpec((NC, BM, H), lambda i: (0, i, 0)),
    ],
    out_shape=[
        jax.ShapeDtypeStruct((N, D), jnp.float32),
        jax.ShapeDtypeStruct((N, D), jnp.float32),
        jax.ShapeDtypeStruct((NC, N, H), jnp.float32),
    ],
)


def _fin_body(x0_ref, h_ref, xlin_ref, accp_ref, dinv_ref, out_ref):
    dinv = dinv_ref[...]
    acc = jnp.concatenate([accp_ref[0], accp_ref[1]], axis=1)
    conv = dinv * acc + (dinv * dinv) * xlin_ref[...]
    h2 = jnp.where(conv >= 0, conv, SLOPE * conv)
    out_ref[...] = (x0_ref[...] + h_ref[...] + h2) * (1.0 / 3.0)


_fin = pl.pallas_call(
    _fin_body,
    grid=(N // BM,),
    in_specs=[
        pl.BlockSpec((BM, D), lambda i: (i, 0)),
        pl.BlockSpec((BM, D), lambda i: (i, 0)),
        pl.BlockSpec((BM, D), lambda i: (i, 0)),
        pl.BlockSpec((NC, BM, H), lambda i: (0, i, 0)),
        pl.BlockSpec((BM, 1), lambda i: (i, 0)),
    ],
    out_specs=pl.BlockSpec((BM, D), lambda i: (i, 0)),
    out_shape=jax.ShapeDtypeStruct((N, D), jnp.float32),
)


def kernel(poi_embs, edge_index, edge_attr, W0, b0, W1, b1):
    row = edge_index[0]
    col = edge_index[1]
    pe = EPAD - E
    row_p = jnp.concatenate([row, jnp.zeros((pe,), jnp.int32)])
    col_main = jnp.concatenate([col, jnp.zeros((pe,), jnp.int32)]).reshape(EPAD // K, K)
    col_deg = jnp.concatenate([col, jnp.full((pe,), NPAD - 1, jnp.int32)]).reshape(EPAD // K, K)
    b0r = b0.reshape(1, D)
    b1r = b1.reshape(1, D)

    deg, dist = _pre_kernel(col_deg, edge_attr)
    dist_p = jnp.concatenate([dist, jnp.zeros((pe,), jnp.float32)])
    meta = jnp.stack([row_p.reshape(EPAD // K, K),
                      col_main.reshape(EPAD // K, K),
                      lax.bitcast_convert_type(dist_p, jnp.int32).reshape(EPAD // K, K)],
                     axis=1)
    degc = deg[:N].reshape(N, 1)

    dinv, xlin0, xsp0 = _mm0(degc, poi_embs, W0, b0r)
    acc0 = _scatter_kernel(xsp0.reshape(NC * N, H), meta)
    h1, xlin1, xsp1 = _mid(xlin0, acc0.reshape(NC, NPAD, H), dinv, W1, b1r)
    acc1 = _scatter_kernel(xsp1.reshape(NC * N, H), meta)
    return _fin(poi_embs, h1, xlin1, acc1.reshape(NC, NPAD, H), dinv)
